# baseline (speedup 1.0000x reference)
import functools

import jax
import jax.numpy as jnp
from jax import lax
from jax.experimental import pallas as pl
from jax.experimental.pallas import tpu as pltpu

B, P = 4, 2048
N = B * P
SHAPE = 256
GEO = 4
E = 8
LAT = 1024
K = 128
NBLK = (N + E * (K - 1) + K - 1) // K
NPAD = NBLK * K
TW = 272

NEG = -3e38


def _tc_body(be_ref, ts_ref, bid_ref,
             W1_ref, b1_ref, W2_ref, b2_ref, W3_ref, b3_ref,
             C1e_ref, C1g_ref, CB1_ref, C2_ref, CB2_ref, C3_ref, CB3_ref,
             out_ref):
    i = pl.program_id(0)
    codes = ts_ref[:, :SHAPE]
    geo16 = ts_ref[:, SHAPE:SHAPE + 16]

    f32 = jnp.float32
    h = jnp.dot(codes, W1_ref[0], preferred_element_type=f32) + b1_ref[0]
    h = jnp.maximum(h, 0.0)
    h = jnp.dot(h, W2_ref[0], preferred_element_type=f32) + b2_ref[0]
    h = jnp.maximum(h, 0.0)
    enc = jnp.dot(h, W3_ref[0], preferred_element_type=f32) + b3_ref[0]

    t = (jnp.dot(enc, C1e_ref[...], preferred_element_type=f32)
         + jnp.dot(geo16, C1g_ref[...], preferred_element_type=f32)
         + CB1_ref[...])
    t = jnp.maximum(t, 0.0)
    t = jnp.dot(t, C2_ref[...], preferred_element_type=f32) + CB2_ref[...]
    t = jnp.maximum(t, 0.0)
    t = jnp.dot(t, C3_ref[...], preferred_element_type=f32) + CB3_ref[...]

    bid = bid_ref[...]
    mx = [jnp.max(jnp.where(bid == jnp.float32(b), t, NEG), axis=0)
          for b in range(B)]
    res = jnp.stack(mx)

    @pl.when(i == 0)
    def _():
        out_ref[...] = jnp.full((B, LAT), NEG, dtype=jnp.float32)

    out_ref[...] = jnp.maximum(out_ref[...], res)


def _tc_call(be, ts, bidf, W1, b1, W2, b2, W3, b3,
             C1e, C1g, CB1, C2, CB2, C3, CB3):
    grid_spec = pltpu.PrefetchScalarGridSpec(
        num_scalar_prefetch=1,
        grid=(NBLK,),
        in_specs=[
            pl.BlockSpec((K, TW), lambda i, be: (i, 0)),
            pl.BlockSpec((K, 1), lambda i, be: (i, 0)),
            pl.BlockSpec((1, SHAPE, 512), lambda i, be: (be[i], 0, 0)),
            pl.BlockSpec((1, 1, 512), lambda i, be: (be[i], 0, 0)),
            pl.BlockSpec((1, 512, 512), lambda i, be: (be[i], 0, 0)),
            pl.BlockSpec((1, 1, 512), lambda i, be: (be[i], 0, 0)),
            pl.BlockSpec((1, 512, SHAPE), lambda i, be: (be[i], 0, 0)),
            pl.BlockSpec((1, 1, SHAPE), lambda i, be: (be[i], 0, 0)),
            pl.BlockSpec((SHAPE, 512), lambda i, be: (0, 0)),
            pl.BlockSpec((16, 512), lambda i, be: (0, 0)),
            pl.BlockSpec((1, 512), lambda i, be: (0, 0)),
            pl.BlockSpec((512, 1024), lambda i, be: (0, 0)),
            pl.BlockSpec((1, 1024), lambda i, be: (0, 0)),
            pl.BlockSpec((1024, LAT), lambda i, be: (0, 0)),
            pl.BlockSpec((1, LAT), lambda i, be: (0, 0)),
        ],
        out_specs=pl.BlockSpec((B, LAT), lambda i, be: (0, 0)),
    )
    return pl.pallas_call(
        _tc_body,
        grid_spec=grid_spec,
        out_shape=jax.ShapeDtypeStruct((B, LAT), jnp.float32),
    )(be, ts, bidf, W1, b1, W2, b2, W3, b3,
      C1e, C1g, CB1, C2, CB2, C3, CB3)


def kernel(x, cats, W1, b1, W2, b2, W3, b3, CW1, CB1, CW2, CB2, CW3, CB3):
    xt = jnp.transpose(x, (0, 2, 1)).reshape(N, GEO + SHAPE)
    table = jnp.concatenate(
        [xt[:, GEO:], xt[:, :GEO],
         jnp.zeros((N, TW - SHAPE - GEO), jnp.float32)], axis=1)

    cf = cats.reshape(-1)
    onehot = (cf[:, None] == jnp.arange(E, dtype=cf.dtype)).astype(jnp.int32)
    counts = jnp.sum(onehot, axis=0)
    rank = jnp.take_along_axis(jnp.cumsum(onehot, axis=0) - onehot,
                               cf[:, None].astype(jnp.int32), axis=1)[:, 0]
    padded = ((counts + K - 1) // K) * K
    ends = jnp.cumsum(padded)
    off = ends - padded
    dest = off[cf] + rank
    src = jnp.arange(N, dtype=jnp.int32)
    gidx = jnp.zeros((NPAD,), jnp.int32).at[dest].set(src)
    bidf = jnp.full((NPAD,), -1.0, jnp.float32).at[dest].set(
        (src // P).astype(jnp.float32)).reshape(NPAD, 1)
    bstart = jnp.arange(NBLK, dtype=jnp.int32) * K
    be = jnp.minimum(
        jnp.sum((bstart[:, None] >= ends[None, :]).astype(jnp.int32), axis=1),
        E - 1).astype(jnp.int32)

    ts = jnp.take(table, gidx, axis=0)

    C1g = jnp.concatenate(
        [CW1[:GEO], jnp.zeros((16 - GEO, 512), jnp.float32)], axis=0)
    out = _tc_call(
        be, ts, bidf,
        W1, b1.reshape(E, 1, 512), W2, b2.reshape(E, 1, 512),
        W3, b3.reshape(E, 1, SHAPE),
        CW1[GEO:], C1g, CB1.reshape(1, 512),
        CW2, CB2.reshape(1, 1024), CW3, CB3.reshape(1, LAT))
    return out


# speedup vs baseline: 1.1193x; 1.1193x over previous
import jax
import jax.numpy as jnp
from jax import lax
from jax.experimental import pallas as pl
from jax.experimental.pallas import tpu as pltpu
from jax.experimental.pallas import tpu_sc as plsc

B, P = 4, 2048
N = B * P
SHAPE = 256
GEO = 4
E = 8
LAT = 1024
K = 128
NBLK = (N + E * (K - 1) + K - 1) // K
NPAD = NBLK * K

NEG = -3e38

NW = 32
RPW = NPAD // NW
NCH = 3
CH = RPW // NCH


def _sc_gather_body(table_hbm, gidx_hbm, out_hbm, idx_v, rows_v, sem):
    wid = lax.axis_index("s") * 2 + lax.axis_index("c")
    base = wid * RPW
    pltpu.sync_copy(gidx_hbm.at[wid], idx_v)
    copies = [pltpu.async_copy(table_hbm.at[idx_v.at[j]], rows_v.at[j], sem)
              for j in range(NCH)]
    for c in copies:
        c.wait()
    for j in range(NCH):
        pltpu.sync_copy(rows_v.at[j], out_hbm.at[pl.ds(base + j * CH, CH)])


def _sc_gather(table, gidx3):
    mesh = plsc.VectorSubcoreMesh(core_axis_name="c", subcore_axis_name="s")
    return pl.kernel(
        _sc_gather_body,
        mesh=mesh,
        out_type=jax.ShapeDtypeStruct((NPAD, SHAPE), jnp.float32),
        scratch_types=[
            pltpu.VMEM((NCH, CH), jnp.int32),
            pltpu.VMEM((NCH, CH, SHAPE), jnp.float32),
            pltpu.SemaphoreType.DMA,
        ],
    )(table, gidx3)


def _tc_body(be_ref, ts_ref, bg_ref,
             W1_ref, b1_ref, W2_ref, b2_ref, W3_ref, b3_ref,
             C1e_ref, C1g_ref, CB1_ref, C2_ref, CB2_ref, C3_ref, CB3_ref,
             out_ref):
    i = pl.program_id(0)
    codes = ts_ref[...]
    bg = bg_ref[...]

    f32 = jnp.float32
    h = jnp.dot(codes, W1_ref[0], preferred_element_type=f32) + b1_ref[0]
    h = jnp.maximum(h, 0.0)
    h = jnp.dot(h, W2_ref[0], preferred_element_type=f32) + b2_ref[0]
    h = jnp.maximum(h, 0.0)
    enc = jnp.dot(h, W3_ref[0], preferred_element_type=f32) + b3_ref[0]

    t = (jnp.dot(enc, C1e_ref[...], preferred_element_type=f32)
         + jnp.dot(bg, C1g_ref[...], preferred_element_type=f32)
         + CB1_ref[...])
    t = jnp.maximum(t, 0.0)
    t = jnp.dot(t, C2_ref[...], preferred_element_type=f32) + CB2_ref[...]
    t = jnp.maximum(t, 0.0)
    t = jnp.dot(t, C3_ref[...], preferred_element_type=f32) + CB3_ref[...]

    bid = bg[:, :1]
    mx = [jnp.max(jnp.where(bid == jnp.float32(b), t, NEG), axis=0)
          for b in range(B)]
    res = jnp.stack(mx)

    @pl.when(i == 0)
    def _():
        out_ref[...] = jnp.full((B, LAT), NEG, dtype=jnp.float32)

    out_ref[...] = jnp.maximum(out_ref[...], res)


def _tc_call(be, ts, bg, W1, b1, W2, b2, W3, b3,
             C1e, C1g, CB1, C2, CB2, C3, CB3):
    grid_spec = pltpu.PrefetchScalarGridSpec(
        num_scalar_prefetch=1,
        grid=(NBLK,),
        in_specs=[
            pl.BlockSpec((K, SHAPE), lambda i, be: (i, 0)),
            pl.BlockSpec((K, 16), lambda i, be: (i, 0)),
            pl.BlockSpec((1, SHAPE, 512), lambda i, be: (be[i], 0, 0)),
            pl.BlockSpec((1, 1, 512), lambda i, be: (be[i], 0, 0)),
            pl.BlockSpec((1, 512, 512), lambda i, be: (be[i], 0, 0)),
            pl.BlockSpec((1, 1, 512), lambda i, be: (be[i], 0, 0)),
            pl.BlockSpec((1, 512, SHAPE), lambda i, be: (be[i], 0, 0)),
            pl.BlockSpec((1, 1, SHAPE), lambda i, be: (be[i], 0, 0)),
            pl.BlockSpec((SHAPE, 512), lambda i, be: (0, 0)),
            pl.BlockSpec((16, 512), lambda i, be: (0, 0)),
            pl.BlockSpec((1, 512), lambda i, be: (0, 0)),
            pl.BlockSpec((512, 1024), lambda i, be: (0, 0)),
            pl.BlockSpec((1, 1024), lambda i, be: (0, 0)),
            pl.BlockSpec((1024, LAT), lambda i, be: (0, 0)),
            pl.BlockSpec((1, LAT), lambda i, be: (0, 0)),
        ],
        out_specs=pl.BlockSpec((B, LAT), lambda i, be: (0, 0)),
    )
    return pl.pallas_call(
        _tc_body,
        grid_spec=grid_spec,
        out_shape=jax.ShapeDtypeStruct((B, LAT), jnp.float32),
    )(be, ts, bg, W1, b1, W2, b2, W3, b3,
      C1e, C1g, CB1, C2, CB2, C3, CB3)


def kernel(x, cats, W1, b1, W2, b2, W3, b3, CW1, CB1, CW2, CB2, CW3, CB3):
    codes = jnp.transpose(x[:, GEO:, :], (0, 2, 1)).reshape(N, SHAPE)
    geoT = jnp.transpose(x[:, :GEO, :], (0, 2, 1)).reshape(N, GEO)

    cf = cats.reshape(-1)
    onehot = (cf[:, None] == jnp.arange(E, dtype=cf.dtype)).astype(jnp.int32)
    counts = jnp.sum(onehot, axis=0)
    rank = jnp.take_along_axis(jnp.cumsum(onehot, axis=0) - onehot,
                               cf[:, None].astype(jnp.int32), axis=1)[:, 0]
    padded = ((counts + K - 1) // K) * K
    ends = jnp.cumsum(padded)
    off = ends - padded
    dest = off[cf] + rank
    src = jnp.arange(N, dtype=jnp.int32)
    gidx = jnp.zeros((NPAD,), jnp.int32).at[dest].set(src)
    bgrow = jnp.concatenate(
        [(src // P).astype(jnp.float32)[:, None], geoT,
         jnp.zeros((N, 11), jnp.float32)], axis=1)
    bg = jnp.full((NPAD, 16), -1.0, jnp.float32).at[dest].set(bgrow)
    bstart = jnp.arange(NBLK, dtype=jnp.int32) * K
    be = jnp.minimum(
        jnp.sum((bstart[:, None] >= ends[None, :]).astype(jnp.int32), axis=1),
        E - 1).astype(jnp.int32)

    ts = _sc_gather(codes, gidx.reshape(NW, NCH, CH))

    C1g = jnp.concatenate(
        [jnp.zeros((1, 512), jnp.float32), CW1[:GEO],
         jnp.zeros((11, 512), jnp.float32)], axis=0)
    out = _tc_call(
        be, ts, bg,
        W1, b1.reshape(E, 1, 512), W2, b2.reshape(E, 1, 512),
        W3, b3.reshape(E, 1, SHAPE),
        CW1[GEO:], C1g, CB1.reshape(1, 512),
        CW2, CB2.reshape(1, 1024), CW3, CB3.reshape(1, LAT))
    return out


# speedup vs baseline: 1.5616x; 1.3951x over previous
import jax
import jax.numpy as jnp
from jax import lax
from jax.experimental import pallas as pl
from jax.experimental.pallas import tpu as pltpu
from jax.experimental.pallas import tpu_sc as plsc

B, P = 4, 2048
N = B * P
SHAPE = 256
GEO = 4
E = 8
LAT = 1024
K = 128
NBLK = (N + E * (K - 1) + K - 1) // K
NPAD = NBLK * K
TW = 384

NEG = -3e38

NW = 32
RPW = N // NW
NCH = 2
CH = RPW // NCH


def _sc_scatter_body(cg_hbm, dest_hbm, out_hbm, idx_v, rows_v, sem_i, sem_o):
    wid = lax.axis_index("s") * 2 + lax.axis_index("c")
    pltpu.sync_copy(dest_hbm.at[wid], idx_v)
    loads = [pltpu.async_copy(cg_hbm.at[wid * NCH + j], rows_v.at[j], sem_i)
             for j in range(NCH)]
    stores = []
    for j in range(NCH):
        loads[j].wait()
        stores.append(
            pltpu.async_copy(rows_v.at[j], out_hbm.at[idx_v.at[j]], sem_o))
    for c in stores:
        c.wait()


def _sc_scatter(cg3, dest3):
    mesh = plsc.VectorSubcoreMesh(core_axis_name="c", subcore_axis_name="s")
    return pl.kernel(
        _sc_scatter_body,
        mesh=mesh,
        out_type=jax.ShapeDtypeStruct((NPAD, TW), jnp.float32),
        scratch_types=[
            pltpu.VMEM((NCH, CH), jnp.int32),
            pltpu.VMEM((NCH, CH, TW), jnp.float32),
            pltpu.SemaphoreType.DMA,
            pltpu.SemaphoreType.DMA,
        ],
    )(cg3, dest3)


def _tc_body(be_ref, st_ref, en_ref, ts_ref,
             W1_ref, b1_ref, W2_ref, b2_ref, W3_ref, b3_ref,
             C1e_ref, C1g_ref, CB1_ref, C2_ref, CB2_ref, C3_ref, CB3_ref,
             out_ref):
    i = pl.program_id(0)
    codes = ts_ref[:, :SHAPE]
    bg = ts_ref[:, SHAPE:]

    f32 = jnp.float32
    h = jnp.dot(codes, W1_ref[0], preferred_element_type=f32) + b1_ref[0]
    h = jnp.maximum(h, 0.0)
    h = jnp.dot(h, W2_ref[0], preferred_element_type=f32) + b2_ref[0]
    h = jnp.maximum(h, 0.0)
    enc = jnp.dot(h, W3_ref[0], preferred_element_type=f32) + b3_ref[0]

    t = (jnp.dot(enc, C1e_ref[...], preferred_element_type=f32)
         + jnp.dot(bg, C1g_ref[...], preferred_element_type=f32)
         + CB1_ref[...])
    t = jnp.maximum(t, 0.0)
    t = jnp.dot(t, C2_ref[...], preferred_element_type=f32) + CB2_ref[...]
    t = jnp.maximum(t, 0.0)
    t = jnp.dot(t, C3_ref[...], preferred_element_type=f32) + CB3_ref[...]

    iota = lax.broadcasted_iota(jnp.int32, (K, 1), 0)
    mx = [jnp.max(jnp.where((iota >= st_ref[i, b]) & (iota < en_ref[i, b]),
                            t, NEG), axis=0)
          for b in range(B)]
    res = jnp.stack(mx)

    @pl.when(i == 0)
    def _():
        out_ref[...] = jnp.full((B, LAT), NEG, dtype=jnp.float32)

    out_ref[...] = jnp.maximum(out_ref[...], res)


def _tc_call(be, st, en, ts, W1, b1, W2, b2, W3, b3,
             C1e, C1g, CB1, C2, CB2, C3, CB3):
    grid_spec = pltpu.PrefetchScalarGridSpec(
        num_scalar_prefetch=3,
        grid=(NBLK,),
        in_specs=[
            pl.BlockSpec((K, TW), lambda i, be, st, en: (i, 0)),
            pl.BlockSpec((1, SHAPE, 512), lambda i, be, st, en: (be[i], 0, 0)),
            pl.BlockSpec((1, 1, 512), lambda i, be, st, en: (be[i], 0, 0)),
            pl.BlockSpec((1, 512, 512), lambda i, be, st, en: (be[i], 0, 0)),
            pl.BlockSpec((1, 1, 512), lambda i, be, st, en: (be[i], 0, 0)),
            pl.BlockSpec((1, 512, SHAPE), lambda i, be, st, en: (be[i], 0, 0)),
            pl.BlockSpec((1, 1, SHAPE), lambda i, be, st, en: (be[i], 0, 0)),
            pl.BlockSpec((SHAPE, 512), lambda i, be, st, en: (0, 0)),
            pl.BlockSpec((K, 512), lambda i, be, st, en: (0, 0)),
            pl.BlockSpec((1, 512), lambda i, be, st, en: (0, 0)),
            pl.BlockSpec((512, 1024), lambda i, be, st, en: (0, 0)),
            pl.BlockSpec((1, 1024), lambda i, be, st, en: (0, 0)),
            pl.BlockSpec((1024, LAT), lambda i, be, st, en: (0, 0)),
            pl.BlockSpec((1, LAT), lambda i, be, st, en: (0, 0)),
        ],
        out_specs=pl.BlockSpec((B, LAT), lambda i, be, st, en: (0, 0)),
    )
    return pl.pallas_call(
        _tc_body,
        grid_spec=grid_spec,
        out_shape=jax.ShapeDtypeStruct((B, LAT), jnp.float32),
    )(be, st, en, ts, W1, b1, W2, b2, W3, b3,
      C1e, C1g, CB1, C2, CB2, C3, CB3)


def kernel(x, cats, W1, b1, W2, b2, W3, b3, CW1, CB1, CW2, CB2, CW3, CB3):
    xt = jnp.transpose(x, (0, 2, 1)).reshape(N, GEO + SHAPE)
    cg = jnp.concatenate(
        [xt[:, GEO:], xt[:, :GEO],
         jnp.zeros((N, TW - SHAPE - GEO), jnp.float32)], axis=1)

    cf = cats.reshape(-1).astype(jnp.int32)
    oh = (cf[:, None] == jnp.arange(E, dtype=jnp.int32)).astype(jnp.int32)
    cum = jnp.cumsum(oh, axis=0)
    rank = jnp.sum((cum - oh) * oh, axis=1)
    counts = cum[-1]
    padded = ((counts + K - 1) // K) * K
    ends = jnp.cumsum(padded)
    off = ends - padded
    dest = jnp.sum(oh * off[None, :], axis=1) + rank

    ceb = jnp.sum(oh.reshape(B, P, E), axis=1)
    seb = jnp.cumsum(ceb, axis=0) - ceb

    bstart = jnp.arange(NBLK, dtype=jnp.int32) * K
    be = jnp.minimum(
        jnp.sum((bstart[:, None] >= ends[None, :]).astype(jnp.int32), axis=1),
        E - 1).astype(jnp.int32)
    beoh = (be[:, None] == jnp.arange(E, dtype=jnp.int32)).astype(jnp.int32)
    r0 = bstart - jnp.sum(beoh * off[None, :], axis=1)
    s_sel = jnp.sum(beoh[:, None, :] * seb[None, :, :], axis=2)
    c_sel = jnp.sum(beoh[:, None, :] * ceb[None, :, :], axis=2)
    st = jnp.clip(s_sel - r0[:, None], 0, K).astype(jnp.int32)
    en = jnp.clip(s_sel + c_sel - r0[:, None], 0, K).astype(jnp.int32)

    ts = _sc_scatter(cg.reshape(NW * NCH, CH, TW),
                     dest.reshape(NW, NCH, CH))

    C1g = jnp.concatenate(
        [CW1[:GEO], jnp.zeros((K - GEO, 512), jnp.float32)], axis=0)
    out = _tc_call(
        be, st, en, ts,
        W1, b1.reshape(E, 1, 512), W2, b2.reshape(E, 1, 512),
        W3, b3.reshape(E, 1, SHAPE),
        CW1[GEO:], C1g, CB1.reshape(1, 512),
        CW2, CB2.reshape(1, 1024), CW3, CB3.reshape(1, LAT))
    return out
